# Initial kernel scaffold; baseline (speedup 1.0000x reference)
import jax
import jax.numpy as jnp
from jax.experimental import pallas as pl


def kernel(tokens, table):
    raise NotImplementedError("write your pallas kernel here")


# speedup vs baseline: 13.5727x; 13.5727x over previous
import functools

import jax
import jax.numpy as jnp
from jax import lax
from jax.experimental import pallas as pl
from jax.experimental.pallas import tpu as pltpu
from jax.experimental.pallas import tpu_sc as plsc

_LANES = 16
_CHUNK = 2


def _bow_mean_sc(tokens_flat, table, B, S):
    V, D = table.shape
    info = plsc.get_sparse_core_info()
    NC, NS = info.num_cores, info.num_subcores
    NW = NC * NS
    b_per_w = B // NW
    n_col = D // _LANES
    n_chunks = b_per_w // _CHUNK
    ids_per_chunk = _CHUNK * S
    inv_s = jnp.float32(1.0 / S)

    mesh = plsc.VectorSubcoreMesh(core_axis_name="c", subcore_axis_name="s")

    @functools.partial(
        pl.kernel,
        mesh=mesh,
        out_type=jax.ShapeDtypeStruct((B, D), jnp.float32),
        scratch_types=[
            pltpu.VMEM((ids_per_chunk,), jnp.int32),
            pltpu.VMEM((ids_per_chunk,), jnp.int32),
            pltpu.VMEM((ids_per_chunk, D), jnp.float32),
            pltpu.VMEM((ids_per_chunk, D), jnp.float32),
            pltpu.VMEM((b_per_w, D), jnp.float32),
            pltpu.SemaphoreType.DMA,
            pltpu.SemaphoreType.DMA,
        ],
    )
    def k(tok_hbm, table_hbm, out_hbm, idx_a, idx_b, buf_a, buf_b, out_v,
          sem_a, sem_b):
        wid = lax.axis_index("c") * NS + lax.axis_index("s")
        tok_base = wid * b_per_w * S

        def start(c, idx_v, buf, sem):
            pltpu.sync_copy(
                tok_hbm.at[pl.ds(tok_base + c * ids_per_chunk, ids_per_chunk)],
                idx_v)
            pltpu.make_async_copy(table_hbm.at[idx_v], buf, sem).start()

        def wait(buf, sem):
            pltpu.make_async_copy(table_hbm.at[idx_a], buf, sem).wait()

        def accumulate(buf, c):
            def rbody(r, acc):
                new = []
                for m in range(_CHUNK):
                    for j in range(n_col):
                        new.append(acc[m * n_col + j]
                                   + buf[m * S + r, pl.ds(j * _LANES, _LANES)])
                return tuple(new)

            zero = jnp.zeros((_LANES,), jnp.float32)
            acc = lax.fori_loop(0, S, rbody, (zero,) * (n_col * _CHUNK))
            for m in range(_CHUNK):
                for j in range(n_col):
                    out_v[c * _CHUNK + m, pl.ds(j * _LANES, _LANES)] = (
                        acc[m * n_col + j] * inv_s)

        start(0, idx_a, buf_a, sem_a)

        def pair(i, carry):
            c0 = i * 2
            start(c0 + 1, idx_b, buf_b, sem_b)
            wait(buf_a, sem_a)
            accumulate(buf_a, c0)

            @pl.when(c0 + 2 < n_chunks)
            def _():
                start(c0 + 2, idx_a, buf_a, sem_a)

            wait(buf_b, sem_b)
            accumulate(buf_b, c0 + 1)
            return carry

        lax.fori_loop(0, n_chunks // 2, pair, 0)

        pltpu.sync_copy(out_v, out_hbm.at[pl.ds(wid * b_per_w, b_per_w)])

    return k(tokens_flat, table)


def kernel(tokens, table):
    B, S = tokens.shape
    tokens_flat = tokens.astype(jnp.int32).reshape(B * S)
    table = table.astype(jnp.float32)
    return _bow_mean_sc(tokens_flat, table, B, S)
